# baseline (speedup 1.0000x reference)
import functools

import jax
import jax.numpy as jnp
from jax import lax
from jax.experimental import pallas as pl
from jax.experimental.pallas import tpu as pltpu
from jax.experimental.pallas import tpu_sc as plsc

N = 10000
D = 128
NC, NS = 2, 16
NW = NC * NS
CH = 128
NPAD = 10112
RPT = NPAD // NS
BLK = 2000


SUPER = 8


def _pad_to(e):
    q = NW * CH * SUPER
    return ((e + q - 1) // q) * q


def _make_scatter_kernel(eps):
    mesh = plsc.VectorSubcoreMesh(core_axis_name="c", subcore_axis_name="s",
                                  num_cores=NC, num_subcores=NS)
    k = len(eps)
    out_type = tuple(
        jax.ShapeDtypeStruct((NC * NPAD, D), jnp.float32) for _ in range(k)
    )
    scratch = [
        pltpu.VMEM_SHARED((NPAD, D), jnp.float32),
        pltpu.VMEM((SUPER, CH), jnp.int32),
        pltpu.VMEM((SUPER, CH), jnp.int32),
        pltpu.VMEM((CH, D), jnp.float32),
        pltpu.VMEM((CH, D), jnp.float32),
        pltpu.VMEM((8, D), jnp.float32),
        pltpu.SemaphoreType.DMA,
        pltpu.SemaphoreType.DMA,
        pltpu.SemaphoreType.DMA,
        pltpu.SemaphoreType.DMA,
    ]

    @functools.partial(pl.kernel, out_type=out_type, mesh=mesh,
                       scratch_types=scratch)
    def scat_kernel(*refs):
        gs = refs[:k]
        srcs = refs[k:2 * k]
        dsts = refs[2 * k:3 * k]
        outs = refs[3 * k:4 * k]
        (acc, sidx, didx, rows0, rows1, zbuf,
         gsem0, gsem1, ssem0, ssem1) = refs[4 * k:]
        rows = (rows0, rows1)
        gsems = (gsem0, gsem1)
        ssems = (ssem0, ssem1)
        c = lax.axis_index("c")
        s = lax.axis_index("s")

        @pl.loop(0, 8)
        def _zfill(i):
            @pl.loop(0, D // 16)
            def _zf2(jj, i=i):
                zbuf[i, pl.ds(jj * 16, 16)] = jnp.zeros((16,), jnp.float32)

        w = s * NC + c
        for kk in range(k):
            @pl.loop(0, RPT // 8)
            def _zero(i):
                pltpu.sync_copy(zbuf, acc.at[pl.ds(s * RPT + i * 8, 8)])
            plsc.subcore_barrier()

            epw = eps[kk] // NW
            nch = epw // CH
            nsup = nch // SUPER
            base_w = w * (epw // CH)
            g_hbm, src_hbm, dst_hbm = gs[kk], srcs[kk], dsts[kk]

            @pl.loop(0, nsup)
            def _super(sb, g_hbm=g_hbm, src_hbm=src_hbm, dst_hbm=dst_hbm,
                       base_w=base_w):
                row0 = base_w + sb * SUPER
                pltpu.sync_copy(src_hbm.at[pl.ds(row0, SUPER)], sidx)
                pltpu.sync_copy(dst_hbm.at[pl.ds(row0, SUPER)], didx)
                sdesc = [None, None]
                for j in range(SUPER):
                    b = j % 2
                    if sdesc[b] is not None:
                        sdesc[b].wait()
                    pltpu.async_copy(g_hbm.at[sidx.at[j]], rows[b],
                                     gsems[b]).wait()
                    sdesc[b] = pltpu.async_copy(rows[b], acc.at[didx.at[j]],
                                                ssems[b], add=True)
                for b in range(2):
                    if sdesc[b] is not None:
                        sdesc[b].wait()

            plsc.subcore_barrier()
            pltpu.sync_copy(acc.at[pl.ds(s * RPT, RPT)],
                            outs[kk].at[pl.ds(c * NPAD + s * RPT, RPT)])
            plsc.subcore_barrier()

    return scat_kernel


def _tc_g_body(x_ref, w_ref, da_ref, db_ref, g_ref):
    deg = da_ref[0:N, 0:1] + db_ref[0:N, 0:1]
    dinv = jnp.where(deg > 0.0, lax.rsqrt(deg), 0.0)
    h = jnp.dot(x_ref[...], w_ref[...], preferred_element_type=jnp.float32)
    g_ref[...] = h * dinv


def _tc_g(x, w, da, db):
    return pl.pallas_call(
        _tc_g_body,
        out_shape=jax.ShapeDtypeStruct((N, D), jnp.float32),
    )(x, w, da, db)


def _tc_post_body(pa, pb, da, db, b, a, fw, fb, e_ref, s_ref):
    deg = da[:, 0:1] + db[:, 0:1]
    dinv = jnp.where(deg > 0.0, lax.rsqrt(deg), 0.0)
    pre = (pa[...] + pb[...]) * dinv + b[...]
    e = jnp.where(pre >= 0.0, pre, a[...] * pre)
    e_ref[...] = e
    t = jnp.tanh(jnp.dot(e, fw[...], preferred_element_type=jnp.float32) + fb[...])
    part = jnp.sum(t, axis=0, keepdims=True)

    @pl.when(pl.program_id(0) == 0)
    def _init():
        s_ref[...] = jnp.zeros_like(s_ref)

    s_ref[...] += jnp.broadcast_to(part, (8, D))


def _tc_post(pa, pb, da, db, b2, a2, fw_t, fb2):
    grid = (N // BLK,)
    return pl.pallas_call(
        _tc_post_body,
        grid=grid,
        in_specs=[
            pl.BlockSpec((BLK, D), lambda i: (i, 0)),
            pl.BlockSpec((BLK, D), lambda i: (i, 0)),
            pl.BlockSpec((BLK, D), lambda i: (i, 0)),
            pl.BlockSpec((BLK, D), lambda i: (i, 0)),
            pl.BlockSpec((1, D), lambda i: (0, 0)),
            pl.BlockSpec((1, D), lambda i: (0, 0)),
            pl.BlockSpec((D, D), lambda i: (0, 0)),
            pl.BlockSpec((1, D), lambda i: (0, 0)),
        ],
        out_specs=[
            pl.BlockSpec((BLK, D), lambda i: (i, 0)),
            pl.BlockSpec((8, D), lambda i: (0, 0)),
        ],
        out_shape=[
            jax.ShapeDtypeStruct((N, D), jnp.float32),
            jax.ShapeDtypeStruct((8, D), jnp.float32),
        ],
    )(pa[:N], pb[:N], da[:N], db[:N], b2, a2, fw_t, fb2)


def _tc_post_host_body(pa, pb, da, db, b, a, e_ref):
    deg = da[:, 0:1] + db[:, 0:1]
    dinv = jnp.where(deg > 0.0, lax.rsqrt(deg), 0.0)
    pre = (pa[...] + pb[...]) * dinv + b[...]
    e_ref[...] = jnp.where(pre >= 0.0, pre, a[...] * pre)


def _tc_post_host(pa, pb, da, db, b2, a2):
    grid = (N // BLK,)
    return pl.pallas_call(
        _tc_post_host_body,
        grid=grid,
        in_specs=[
            pl.BlockSpec((BLK, D), lambda i: (i, 0)),
            pl.BlockSpec((BLK, D), lambda i: (i, 0)),
            pl.BlockSpec((BLK, D), lambda i: (i, 0)),
            pl.BlockSpec((BLK, D), lambda i: (i, 0)),
            pl.BlockSpec((1, D), lambda i: (0, 0)),
            pl.BlockSpec((1, D), lambda i: (0, 0)),
        ],
        out_specs=pl.BlockSpec((BLK, D), lambda i: (i, 0)),
        out_shape=jax.ShapeDtypeStruct((N, D), jnp.float32),
    )(pa[:N], pb[:N], da[:N], db[:N], b2, a2)


def _tc_combine_body(e0, e1, s0, s1, att, z_ref):
    w0 = jnp.sum(att[...] * s0[0:1, :]) * (1.0 / N)
    w1 = jnp.sum(att[...] * s1[0:1, :]) * (1.0 / N)
    m = jnp.maximum(w0, w1)
    x0 = jnp.exp(w0 - m)
    x1 = jnp.exp(w1 - m)
    inv = 1.0 / (x0 + x1)
    z_ref[...] = (x0 * inv) * e0[...] + (x1 * inv) * e1[...]


def _tc_combine(e0, e1, s0, s1, att2):
    grid = (N // BLK,)
    return pl.pallas_call(
        _tc_combine_body,
        grid=grid,
        in_specs=[
            pl.BlockSpec((BLK, D), lambda i: (i, 0)),
            pl.BlockSpec((BLK, D), lambda i: (i, 0)),
            pl.BlockSpec((8, D), lambda i: (0, 0)),
            pl.BlockSpec((8, D), lambda i: (0, 0)),
            pl.BlockSpec((1, D), lambda i: (0, 0)),
        ],
        out_specs=pl.BlockSpec((BLK, D), lambda i: (i, 0)),
        out_shape=jax.ShapeDtypeStruct((N, D), jnp.float32),
    )(e0, e1, s0, s1, att2)


_E_LIST = (160000, 320000, 320000, 320000, 320000)
_EP_LIST = tuple(_pad_to(e) for e in _E_LIST)
_make_scatter_kernel = functools.lru_cache(maxsize=None)(_make_scatter_kernel)


def _pad_edges(ei, ep):
    e = ei.shape[1]
    src = jnp.concatenate([ei[0], jnp.zeros((ep - e,), jnp.int32)])
    dst = jnp.concatenate([ei[1], jnp.full((ep - e,), N, jnp.int32)])
    return src.reshape(ep // CH, CH), dst.reshape(ep // CH, CH)


def kernel(x_host, x_vm, x_instance, ei_host_dc, ei_vm_dc, ei_vm_host,
           ei_inst_task, ei_inst_vm,
           W_host_dc, b_host_dc, W_vm_dc, b_vm_dc, W_vm_host, b_vm_host,
           W_inst_task, b_inst_task, W_inst_vm, b_inst_vm, prelu_a,
           fcW_host, fcb_host, att_host, fcW_vm, fcb_vm, att_vm,
           fcW_inst, fcb_inst, att_inst):
    eis = (ei_host_dc, ei_vm_dc, ei_vm_host, ei_inst_task, ei_inst_vm)
    xs = (x_host, x_vm, x_vm, x_instance, x_instance)
    Ws = (W_host_dc, W_vm_dc, W_vm_host, W_inst_task, W_inst_vm)
    bs = (b_host_dc, b_vm_dc, b_vm_host, b_inst_task, b_inst_vm)

    padded = [_pad_edges(ei, ep) for ei, ep in zip(eis, _EP_LIST)]
    ones_tab = jnp.ones((N, D), jnp.float32)
    zsrcs = [jnp.zeros((ep // CH, CH), jnp.int32) for ep in _EP_LIST]
    scat = _make_scatter_kernel(_EP_LIST)
    deg_flat = scat(*([ones_tab] * 5), *zsrcs, *(d for _, d in padded))
    deg_parts = []
    for t in deg_flat:
        deg_parts.extend((t[:NPAD], t[NPAD:]))

    a2 = jnp.broadcast_to(prelu_a.reshape(1, 1), (1, D))

    g_list = [
        _tc_g(xs[mp], Ws[mp], deg_parts[2 * mp], deg_parts[2 * mp + 1])
        for mp in range(5)
    ]
    part_flat = scat(*g_list, *(s for s, _ in padded), *(d for _, d in padded))
    parts = []
    for t in part_flat:
        parts.extend((t[:NPAD], t[NPAD:]))

    es = []
    ssums = []
    for mp in range(5):
        da, db = deg_parts[2 * mp], deg_parts[2 * mp + 1]
        pa, pb = parts[2 * mp], parts[2 * mp + 1]
        b2 = bs[mp].reshape(1, D)
        if mp == 0:
            es.append(_tc_post_host(pa, pb, da, db, b2, a2))
            ssums.append(None)
        else:
            fw_t, fb2, _ = _ATT_PARAMS(mp, fcW_vm, fcb_vm, fcW_inst, fcb_inst)
            e, ss = _tc_post(pa, pb, da, db, b2, a2, fw_t, fb2)
            es.append(e)
            ssums.append(ss)

    host_z = es[0]
    vm_z = _tc_combine(es[1], es[2], ssums[1], ssums[2], att_vm)
    inst_z = _tc_combine(es[3], es[4], ssums[3], ssums[4], att_inst)
    return (host_z, vm_z, inst_z)


def _ATT_PARAMS(mp, fcW_vm, fcb_vm, fcW_inst, fcb_inst):
    if mp in (1, 2):
        return fcW_vm.T, fcb_vm.reshape(1, D), None
    return fcW_inst.T, fcb_inst.reshape(1, D), None


# speedup vs baseline: 12.0053x; 12.0053x over previous
import functools

import jax
import jax.numpy as jnp
from jax import lax
from jax.experimental import pallas as pl
from jax.experimental.pallas import tpu as pltpu
from jax.experimental.pallas import tpu_sc as plsc

N = 10000
D = 128
NC, NS = 2, 16
NW = NC * NS
CH = 128
NPAD = 10112
RPT = NPAD // NS
BLK = 2000


SUPER = 8


def _pad_to(e):
    q = NW * CH * SUPER
    return ((e + q - 1) // q) * q


def _make_scatter_kernel(eps):
    mesh = plsc.VectorSubcoreMesh(core_axis_name="c", subcore_axis_name="s",
                                  num_cores=NC, num_subcores=NS)
    k = len(eps)
    out_type = tuple(
        jax.ShapeDtypeStruct((NC * NPAD, D), jnp.float32) for _ in range(k)
    )
    scratch = [
        pltpu.VMEM_SHARED((NPAD, D), jnp.float32),
        pltpu.VMEM((SUPER, CH), jnp.int32),
        pltpu.VMEM((SUPER, CH), jnp.int32),
        pltpu.VMEM((CH, D), jnp.float32),
        pltpu.VMEM((CH, D), jnp.float32),
        pltpu.VMEM((8, D), jnp.float32),
        pltpu.SemaphoreType.DMA,
        pltpu.SemaphoreType.DMA,
        pltpu.SemaphoreType.DMA,
        pltpu.SemaphoreType.DMA,
    ]

    @functools.partial(pl.kernel, out_type=out_type, mesh=mesh,
                       scratch_types=scratch)
    def scat_kernel(*refs):
        gs = refs[:k]
        srcs = refs[k:2 * k]
        dsts = refs[2 * k:3 * k]
        outs = refs[3 * k:4 * k]
        (acc, sidx, didx, rows0, rows1, zbuf,
         gsem0, gsem1, ssem0, ssem1) = refs[4 * k:]
        rows = (rows0, rows1)
        gsems = (gsem0, gsem1)
        ssems = (ssem0, ssem1)
        c = lax.axis_index("c")
        s = lax.axis_index("s")

        @pl.loop(0, 8)
        def _zfill(i):
            @pl.loop(0, D // 16)
            def _zf2(jj, i=i):
                zbuf[i, pl.ds(jj * 16, 16)] = jnp.zeros((16,), jnp.float32)

        w = s * NC + c
        for kk in range(k):
            @pl.loop(0, RPT // 8)
            def _zero(i):
                pltpu.sync_copy(zbuf, acc.at[pl.ds(s * RPT + i * 8, 8)])
            plsc.subcore_barrier()

            epw = eps[kk] // NW
            nch = epw // CH
            nsup = nch // SUPER
            base_w = w * (epw // CH)
            g_hbm, src_hbm, dst_hbm = gs[kk], srcs[kk], dsts[kk]

            @pl.loop(0, nsup)
            def _super(sb, g_hbm=g_hbm, src_hbm=src_hbm, dst_hbm=dst_hbm,
                       base_w=base_w):
                row0 = base_w + sb * SUPER
                pltpu.sync_copy(src_hbm.at[pl.ds(row0, SUPER)], sidx)
                pltpu.sync_copy(dst_hbm.at[pl.ds(row0, SUPER)], didx)
                sdesc = [None, None]
                for j in range(SUPER):
                    b = j % 2
                    if sdesc[b] is not None:
                        sdesc[b].wait()
                    pltpu.async_copy(g_hbm.at[sidx.at[j]], rows[b],
                                     gsems[b]).wait()
                    sdesc[b] = pltpu.async_copy(rows[b], acc.at[didx.at[j]],
                                                ssems[b], add=True)
                for b in range(2):
                    if sdesc[b] is not None:
                        sdesc[b].wait()

            plsc.subcore_barrier()
            pltpu.sync_copy(acc.at[pl.ds(s * RPT, RPT)],
                            outs[kk].at[pl.ds(c * NPAD + s * RPT, RPT)])
            plsc.subcore_barrier()

    return scat_kernel


def _tc_g_body(x_ref, w_ref, da_ref, db_ref, g_ref):
    deg = da_ref[0:N, 0:1] + db_ref[0:N, 0:1]
    dinv = jnp.where(deg > 0.0, lax.rsqrt(deg), 0.0)
    h = jnp.dot(x_ref[...], w_ref[...], preferred_element_type=jnp.float32)
    g_ref[...] = h * dinv


def _tc_g(x, w, da, db):
    return pl.pallas_call(
        _tc_g_body,
        out_shape=jax.ShapeDtypeStruct((N, D), jnp.float32),
    )(x, w, da, db)


def _tc_post_body(pa, pb, da, db, b, a, fw, fb, e_ref, s_ref):
    deg = da[:, 0:1] + db[:, 0:1]
    dinv = jnp.where(deg > 0.0, lax.rsqrt(deg), 0.0)
    pre = (pa[...] + pb[...]) * dinv + b[...]
    e = jnp.where(pre >= 0.0, pre, a[...] * pre)
    e_ref[...] = e
    t = jnp.tanh(jnp.dot(e, fw[...], preferred_element_type=jnp.float32) + fb[...])
    part = jnp.sum(t, axis=0, keepdims=True)

    @pl.when(pl.program_id(0) == 0)
    def _init():
        s_ref[...] = jnp.zeros_like(s_ref)

    s_ref[...] += jnp.broadcast_to(part, (8, D))


def _tc_post(pa, pb, da, db, b2, a2, fw_t, fb2):
    grid = (N // BLK,)
    return pl.pallas_call(
        _tc_post_body,
        grid=grid,
        in_specs=[
            pl.BlockSpec((BLK, D), lambda i: (i, 0)),
            pl.BlockSpec((BLK, D), lambda i: (i, 0)),
            pl.BlockSpec((BLK, D), lambda i: (i, 0)),
            pl.BlockSpec((BLK, D), lambda i: (i, 0)),
            pl.BlockSpec((1, D), lambda i: (0, 0)),
            pl.BlockSpec((1, D), lambda i: (0, 0)),
            pl.BlockSpec((D, D), lambda i: (0, 0)),
            pl.BlockSpec((1, D), lambda i: (0, 0)),
        ],
        out_specs=[
            pl.BlockSpec((BLK, D), lambda i: (i, 0)),
            pl.BlockSpec((8, D), lambda i: (0, 0)),
        ],
        out_shape=[
            jax.ShapeDtypeStruct((N, D), jnp.float32),
            jax.ShapeDtypeStruct((8, D), jnp.float32),
        ],
    )(pa[:N], pb[:N], da[:N], db[:N], b2, a2, fw_t, fb2)


def _tc_post_host_body(pa, pb, da, db, b, a, e_ref):
    deg = da[:, 0:1] + db[:, 0:1]
    dinv = jnp.where(deg > 0.0, lax.rsqrt(deg), 0.0)
    pre = (pa[...] + pb[...]) * dinv + b[...]
    e_ref[...] = jnp.where(pre >= 0.0, pre, a[...] * pre)


def _tc_post_host(pa, pb, da, db, b2, a2):
    grid = (N // BLK,)
    return pl.pallas_call(
        _tc_post_host_body,
        grid=grid,
        in_specs=[
            pl.BlockSpec((BLK, D), lambda i: (i, 0)),
            pl.BlockSpec((BLK, D), lambda i: (i, 0)),
            pl.BlockSpec((BLK, D), lambda i: (i, 0)),
            pl.BlockSpec((BLK, D), lambda i: (i, 0)),
            pl.BlockSpec((1, D), lambda i: (0, 0)),
            pl.BlockSpec((1, D), lambda i: (0, 0)),
        ],
        out_specs=pl.BlockSpec((BLK, D), lambda i: (i, 0)),
        out_shape=jax.ShapeDtypeStruct((N, D), jnp.float32),
    )(pa[:N], pb[:N], da[:N], db[:N], b2, a2)


def _tc_combine_body(e0, e1, s0, s1, att, z_ref):
    w0 = jnp.sum(att[...] * s0[0:1, :]) * (1.0 / N)
    w1 = jnp.sum(att[...] * s1[0:1, :]) * (1.0 / N)
    m = jnp.maximum(w0, w1)
    x0 = jnp.exp(w0 - m)
    x1 = jnp.exp(w1 - m)
    inv = 1.0 / (x0 + x1)
    z_ref[...] = (x0 * inv) * e0[...] + (x1 * inv) * e1[...]


def _tc_combine(e0, e1, s0, s1, att2):
    grid = (N // BLK,)
    return pl.pallas_call(
        _tc_combine_body,
        grid=grid,
        in_specs=[
            pl.BlockSpec((BLK, D), lambda i: (i, 0)),
            pl.BlockSpec((BLK, D), lambda i: (i, 0)),
            pl.BlockSpec((8, D), lambda i: (0, 0)),
            pl.BlockSpec((8, D), lambda i: (0, 0)),
            pl.BlockSpec((1, D), lambda i: (0, 0)),
        ],
        out_specs=pl.BlockSpec((BLK, D), lambda i: (i, 0)),
        out_shape=jax.ShapeDtypeStruct((N, D), jnp.float32),
    )(e0, e1, s0, s1, att2)


_E_LIST = (160000, 320000, 320000, 320000, 320000)
_EP_LIST = tuple(_pad_to(e) for e in _E_LIST)
_make_scatter_kernel = functools.lru_cache(maxsize=None)(_make_scatter_kernel)


def _pad_edges(ei, ep):
    e = ei.shape[1]
    src = jnp.concatenate([ei[0], jnp.zeros((ep - e,), jnp.int32)])
    dst = jnp.concatenate([ei[1], jnp.full((ep - e,), N, jnp.int32)])
    return src.reshape(ep // CH, CH), dst.reshape(ep // CH, CH)


def kernel(x_host, x_vm, x_instance, ei_host_dc, ei_vm_dc, ei_vm_host,
           ei_inst_task, ei_inst_vm,
           W_host_dc, b_host_dc, W_vm_dc, b_vm_dc, W_vm_host, b_vm_host,
           W_inst_task, b_inst_task, W_inst_vm, b_inst_vm, prelu_a,
           fcW_host, fcb_host, att_host, fcW_vm, fcb_vm, att_vm,
           fcW_inst, fcb_inst, att_inst):
    eis = (ei_host_dc, ei_vm_dc, ei_vm_host, ei_inst_task, ei_inst_vm)
    xs = (x_host, x_vm, x_vm, x_instance, x_instance)
    Ws = (W_host_dc, W_vm_dc, W_vm_host, W_inst_task, W_inst_vm)
    bs = (b_host_dc, b_vm_dc, b_vm_host, b_inst_task, b_inst_vm)

    padded = [_pad_edges(ei, ep) for ei, ep in zip(eis, _EP_LIST)]
    ones_tab = jnp.ones((N, D), jnp.float32)
    scat = _make_scatter_kernel(_EP_LIST)
    dpad = [jnp.where(d < N, d, 0) for _, d in padded]
    deg_flat = scat(*([ones_tab] * 5), *dpad, *(d for _, d in padded))
    deg_parts = []
    for t in deg_flat:
        deg_parts.extend((t[:NPAD], t[NPAD:]))

    a2 = jnp.broadcast_to(prelu_a.reshape(1, 1), (1, D))

    g_list = [
        _tc_g(xs[mp], Ws[mp], deg_parts[2 * mp], deg_parts[2 * mp + 1])
        for mp in range(5)
    ]
    part_flat = scat(*g_list, *(s for s, _ in padded), *(d for _, d in padded))
    parts = []
    for t in part_flat:
        parts.extend((t[:NPAD], t[NPAD:]))

    es = []
    ssums = []
    for mp in range(5):
        da, db = deg_parts[2 * mp], deg_parts[2 * mp + 1]
        pa, pb = parts[2 * mp], parts[2 * mp + 1]
        b2 = bs[mp].reshape(1, D)
        if mp == 0:
            es.append(_tc_post_host(pa, pb, da, db, b2, a2))
            ssums.append(None)
        else:
            fw_t, fb2, _ = _ATT_PARAMS(mp, fcW_vm, fcb_vm, fcW_inst, fcb_inst)
            e, ss = _tc_post(pa, pb, da, db, b2, a2, fw_t, fb2)
            es.append(e)
            ssums.append(ss)

    host_z = es[0]
    vm_z = _tc_combine(es[1], es[2], ssums[1], ssums[2], att_vm)
    inst_z = _tc_combine(es[3], es[4], ssums[3], ssums[4], att_inst)
    return (host_z, vm_z, inst_z)


def _ATT_PARAMS(mp, fcW_vm, fcb_vm, fcW_inst, fcb_inst):
    if mp in (1, 2):
        return fcW_vm.T, fcb_vm.reshape(1, D), None
    return fcW_inst.T, fcb_inst.reshape(1, D), None


# speedup vs baseline: 12.3952x; 1.0325x over previous
import functools

import jax
import jax.numpy as jnp
from jax import lax
from jax.experimental import pallas as pl
from jax.experimental.pallas import tpu as pltpu
from jax.experimental.pallas import tpu_sc as plsc

N = 10000
D = 128
NC, NS = 2, 16
NW = NC * NS
CH = 128
NPAD = 10112
RPT = NPAD // NS
BLK = 2000


SUPER = 8


def _pad_to(e):
    q = NW * CH * SUPER
    return ((e + q - 1) // q) * q


def _make_scatter_kernel(eps):
    mesh = plsc.VectorSubcoreMesh(core_axis_name="c", subcore_axis_name="s",
                                  num_cores=NC, num_subcores=NS)
    k = len(eps)
    out_type = tuple(
        jax.ShapeDtypeStruct((NC * NPAD, D), jnp.float32) for _ in range(k)
    )
    scratch = [
        pltpu.VMEM_SHARED((NPAD, D), jnp.float32),
        pltpu.VMEM((SUPER, CH), jnp.int32),
        pltpu.VMEM((SUPER, CH), jnp.int32),
        pltpu.VMEM((CH, D), jnp.float32),
        pltpu.VMEM((CH, D), jnp.float32),
        pltpu.VMEM((8, D), jnp.float32),
        pltpu.SemaphoreType.DMA,
        pltpu.SemaphoreType.DMA,
        pltpu.SemaphoreType.DMA,
        pltpu.SemaphoreType.DMA,
    ]

    @functools.partial(pl.kernel, out_type=out_type, mesh=mesh,
                       scratch_types=scratch)
    def scat_kernel(*refs):
        gs = refs[:k]
        srcs = refs[k:2 * k]
        dsts = refs[2 * k:3 * k]
        outs = refs[3 * k:4 * k]
        (acc, sidx, didx, rows0, rows1, zbuf,
         gsem0, gsem1, ssem0, ssem1) = refs[4 * k:]
        rows = (rows0, rows1)
        gsems = (gsem0, gsem1)
        ssems = (ssem0, ssem1)
        c = lax.axis_index("c")
        s = lax.axis_index("s")

        @pl.loop(0, 8)
        def _zfill(i):
            @pl.loop(0, D // 16)
            def _zf2(jj, i=i):
                zbuf[i, pl.ds(jj * 16, 16)] = jnp.zeros((16,), jnp.float32)

        w = s * NC + c
        for kk in range(k):
            @pl.loop(0, RPT // 8)
            def _zero(i):
                pltpu.sync_copy(zbuf, acc.at[pl.ds(s * RPT + i * 8, 8)])
            plsc.subcore_barrier()

            epw = eps[kk] // NW
            nch = epw // CH
            nsup = nch // SUPER
            base_w = w * (epw // CH)
            g_hbm, src_hbm, dst_hbm = gs[kk], srcs[kk], dsts[kk]

            @pl.loop(0, nsup)
            def _super(sb, g_hbm=g_hbm, src_hbm=src_hbm, dst_hbm=dst_hbm,
                       base_w=base_w):
                row0 = base_w + sb * SUPER
                pltpu.sync_copy(src_hbm.at[pl.ds(row0, SUPER)], sidx)
                pltpu.sync_copy(dst_hbm.at[pl.ds(row0, SUPER)], didx)
                gdesc = [None, None]
                sdesc = [None, None]
                gdesc[0] = pltpu.async_copy(g_hbm.at[sidx.at[0]], rows[0],
                                            gsems[0])
                for j in range(SUPER):
                    b = j % 2
                    nb = (j + 1) % 2
                    if j + 1 < SUPER:
                        if sdesc[nb] is not None:
                            sdesc[nb].wait()
                        gdesc[nb] = pltpu.async_copy(
                            g_hbm.at[sidx.at[j + 1]], rows[nb], gsems[nb])
                    gdesc[b].wait()
                    sdesc[b] = pltpu.async_copy(rows[b], acc.at[didx.at[j]],
                                                ssems[b], add=True)
                for b in range(2):
                    if sdesc[b] is not None:
                        sdesc[b].wait()

            plsc.subcore_barrier()
            pltpu.sync_copy(acc.at[pl.ds(s * RPT, RPT)],
                            outs[kk].at[pl.ds(c * NPAD + s * RPT, RPT)])
            plsc.subcore_barrier()

    return scat_kernel


def _tc_g_body(x_ref, w_ref, da_ref, db_ref, g_ref):
    deg = da_ref[0:N, 0:1] + db_ref[0:N, 0:1]
    dinv = jnp.where(deg > 0.0, lax.rsqrt(deg), 0.0)
    h = jnp.dot(x_ref[...], w_ref[...], preferred_element_type=jnp.float32)
    g_ref[...] = h * dinv


def _tc_g(x, w, da, db):
    return pl.pallas_call(
        _tc_g_body,
        out_shape=jax.ShapeDtypeStruct((N, D), jnp.float32),
    )(x, w, da, db)


def _tc_post_body(pa, pb, da, db, b, a, fw, fb, e_ref, s_ref):
    deg = da[:, 0:1] + db[:, 0:1]
    dinv = jnp.where(deg > 0.0, lax.rsqrt(deg), 0.0)
    pre = (pa[...] + pb[...]) * dinv + b[...]
    e = jnp.where(pre >= 0.0, pre, a[...] * pre)
    e_ref[...] = e
    t = jnp.tanh(jnp.dot(e, fw[...], preferred_element_type=jnp.float32) + fb[...])
    part = jnp.sum(t, axis=0, keepdims=True)

    @pl.when(pl.program_id(0) == 0)
    def _init():
        s_ref[...] = jnp.zeros_like(s_ref)

    s_ref[...] += jnp.broadcast_to(part, (8, D))


def _tc_post(pa, pb, da, db, b2, a2, fw_t, fb2):
    grid = (N // BLK,)
    return pl.pallas_call(
        _tc_post_body,
        grid=grid,
        in_specs=[
            pl.BlockSpec((BLK, D), lambda i: (i, 0)),
            pl.BlockSpec((BLK, D), lambda i: (i, 0)),
            pl.BlockSpec((BLK, D), lambda i: (i, 0)),
            pl.BlockSpec((BLK, D), lambda i: (i, 0)),
            pl.BlockSpec((1, D), lambda i: (0, 0)),
            pl.BlockSpec((1, D), lambda i: (0, 0)),
            pl.BlockSpec((D, D), lambda i: (0, 0)),
            pl.BlockSpec((1, D), lambda i: (0, 0)),
        ],
        out_specs=[
            pl.BlockSpec((BLK, D), lambda i: (i, 0)),
            pl.BlockSpec((8, D), lambda i: (0, 0)),
        ],
        out_shape=[
            jax.ShapeDtypeStruct((N, D), jnp.float32),
            jax.ShapeDtypeStruct((8, D), jnp.float32),
        ],
    )(pa[:N], pb[:N], da[:N], db[:N], b2, a2, fw_t, fb2)


def _tc_post_host_body(pa, pb, da, db, b, a, e_ref):
    deg = da[:, 0:1] + db[:, 0:1]
    dinv = jnp.where(deg > 0.0, lax.rsqrt(deg), 0.0)
    pre = (pa[...] + pb[...]) * dinv + b[...]
    e_ref[...] = jnp.where(pre >= 0.0, pre, a[...] * pre)


def _tc_post_host(pa, pb, da, db, b2, a2):
    grid = (N // BLK,)
    return pl.pallas_call(
        _tc_post_host_body,
        grid=grid,
        in_specs=[
            pl.BlockSpec((BLK, D), lambda i: (i, 0)),
            pl.BlockSpec((BLK, D), lambda i: (i, 0)),
            pl.BlockSpec((BLK, D), lambda i: (i, 0)),
            pl.BlockSpec((BLK, D), lambda i: (i, 0)),
            pl.BlockSpec((1, D), lambda i: (0, 0)),
            pl.BlockSpec((1, D), lambda i: (0, 0)),
        ],
        out_specs=pl.BlockSpec((BLK, D), lambda i: (i, 0)),
        out_shape=jax.ShapeDtypeStruct((N, D), jnp.float32),
    )(pa[:N], pb[:N], da[:N], db[:N], b2, a2)


def _tc_combine_body(e0, e1, s0, s1, att, z_ref):
    w0 = jnp.sum(att[...] * s0[0:1, :]) * (1.0 / N)
    w1 = jnp.sum(att[...] * s1[0:1, :]) * (1.0 / N)
    m = jnp.maximum(w0, w1)
    x0 = jnp.exp(w0 - m)
    x1 = jnp.exp(w1 - m)
    inv = 1.0 / (x0 + x1)
    z_ref[...] = (x0 * inv) * e0[...] + (x1 * inv) * e1[...]


def _tc_combine(e0, e1, s0, s1, att2):
    grid = (N // BLK,)
    return pl.pallas_call(
        _tc_combine_body,
        grid=grid,
        in_specs=[
            pl.BlockSpec((BLK, D), lambda i: (i, 0)),
            pl.BlockSpec((BLK, D), lambda i: (i, 0)),
            pl.BlockSpec((8, D), lambda i: (0, 0)),
            pl.BlockSpec((8, D), lambda i: (0, 0)),
            pl.BlockSpec((1, D), lambda i: (0, 0)),
        ],
        out_specs=pl.BlockSpec((BLK, D), lambda i: (i, 0)),
        out_shape=jax.ShapeDtypeStruct((N, D), jnp.float32),
    )(e0, e1, s0, s1, att2)


_E_LIST = (160000, 320000, 320000, 320000, 320000)
_EP_LIST = tuple(_pad_to(e) for e in _E_LIST)
_make_scatter_kernel = functools.lru_cache(maxsize=None)(_make_scatter_kernel)


def _pad_edges(ei, ep):
    e = ei.shape[1]
    src = jnp.concatenate([ei[0], jnp.zeros((ep - e,), jnp.int32)])
    dst = jnp.concatenate([ei[1], jnp.full((ep - e,), N, jnp.int32)])
    return src.reshape(ep // CH, CH), dst.reshape(ep // CH, CH)


def kernel(x_host, x_vm, x_instance, ei_host_dc, ei_vm_dc, ei_vm_host,
           ei_inst_task, ei_inst_vm,
           W_host_dc, b_host_dc, W_vm_dc, b_vm_dc, W_vm_host, b_vm_host,
           W_inst_task, b_inst_task, W_inst_vm, b_inst_vm, prelu_a,
           fcW_host, fcb_host, att_host, fcW_vm, fcb_vm, att_vm,
           fcW_inst, fcb_inst, att_inst):
    eis = (ei_host_dc, ei_vm_dc, ei_vm_host, ei_inst_task, ei_inst_vm)
    xs = (x_host, x_vm, x_vm, x_instance, x_instance)
    Ws = (W_host_dc, W_vm_dc, W_vm_host, W_inst_task, W_inst_vm)
    bs = (b_host_dc, b_vm_dc, b_vm_host, b_inst_task, b_inst_vm)

    padded = [_pad_edges(ei, ep) for ei, ep in zip(eis, _EP_LIST)]
    ones_tab = jnp.ones((N, D), jnp.float32)
    scat = _make_scatter_kernel(_EP_LIST)
    dpad = [jnp.where(d < N, d, 0) for _, d in padded]
    deg_flat = scat(*([ones_tab] * 5), *dpad, *(d for _, d in padded))
    deg_parts = []
    for t in deg_flat:
        deg_parts.extend((t[:NPAD], t[NPAD:]))

    a2 = jnp.broadcast_to(prelu_a.reshape(1, 1), (1, D))

    g_list = [
        _tc_g(xs[mp], Ws[mp], deg_parts[2 * mp], deg_parts[2 * mp + 1])
        for mp in range(5)
    ]
    part_flat = scat(*g_list, *(s for s, _ in padded), *(d for _, d in padded))
    parts = []
    for t in part_flat:
        parts.extend((t[:NPAD], t[NPAD:]))

    es = []
    ssums = []
    for mp in range(5):
        da, db = deg_parts[2 * mp], deg_parts[2 * mp + 1]
        pa, pb = parts[2 * mp], parts[2 * mp + 1]
        b2 = bs[mp].reshape(1, D)
        if mp == 0:
            es.append(_tc_post_host(pa, pb, da, db, b2, a2))
            ssums.append(None)
        else:
            fw_t, fb2, _ = _ATT_PARAMS(mp, fcW_vm, fcb_vm, fcW_inst, fcb_inst)
            e, ss = _tc_post(pa, pb, da, db, b2, a2, fw_t, fb2)
            es.append(e)
            ssums.append(ss)

    host_z = es[0]
    vm_z = _tc_combine(es[1], es[2], ssums[1], ssums[2], att_vm)
    inst_z = _tc_combine(es[3], es[4], ssums[3], ssums[4], att_inst)
    return (host_z, vm_z, inst_z)


def _ATT_PARAMS(mp, fcW_vm, fcb_vm, fcW_inst, fcb_inst):
    if mp in (1, 2):
        return fcW_vm.T, fcb_vm.reshape(1, D), None
    return fcW_inst.T, fcb_inst.reshape(1, D), None
